# baseline (speedup 1.0000x reference)
import functools
import jax
import jax.numpy as jnp
from jax import lax
from jax.experimental import pallas as pl
from jax.experimental.pallas import tpu as pltpu
from jax.experimental.pallas import tpu_sc as plsc

NG = 5000
NP = 2000
H = 64
B = 32
EPS = 1e-5
GB = 200
NBLK = NG // GB
NROWS = float(B * NG)

NGP = 5120
NPP = 2048
NTOT = NGP + NPP
DUMP = NTOT - 1
ECO = 100000
EGO = 40000
NW = 32
CHUNK = 128
NCH = 36
EPW = NCH * CHUNK
ETOT = NW * EPW
TPS = NTOT // 16
TW = 128

def _sc_mesh():
    return plsc.VectorSubcoreMesh(core_axis_name="c", subcore_axis_name="s")


@functools.cache
def _make_sc_deg():
    return functools.partial(
        pl.kernel,
        out_type=jax.ShapeDtypeStruct((2, NTOT, TW), jnp.float32),
        mesh=_sc_mesh(),
        scratch_types=[
            pltpu.VMEM((NCH, CHUNK), jnp.int32),
            pltpu.VMEM((NCH, CHUNK), jnp.float32),
            pltpu.VMEM((CHUNK, TW), jnp.float32),
            pltpu.VMEM((CHUNK, TW), jnp.float32),
            pltpu.VMEM_SHARED((NTOT, TW), jnp.float32),
            pltpu.SemaphoreType.DMA,
        ],
    )(_sc_deg_body)


def _sc_deg(col3, ew3, z64):
    return _make_sc_deg()(col3, ew3, z64)


def _sc_deg_body(col_hbm, ew_hbm, z64_hbm, out_hbm,
                 cslab, wslab, s0, s1, acc, semw):
    c = lax.axis_index("c")
    s = lax.axis_index("s")
    w = s * 2 + c
    t0 = pl.multiple_of(s * TPS, 8)

    pltpu.sync_copy(z64_hbm.at[pl.ds(t0, TPS), :], acc.at[pl.ds(t0, TPS), :])
    pltpu.sync_copy(col_hbm.at[w], cslab)
    pltpu.sync_copy(ew_hbm.at[w], wslab)

    def _zsrc(k, cc):
        for g in range(1, TW // 16):
            s0[k, pl.ds(g * 16, 16)] = jnp.zeros((16,), jnp.float32)
            s1[k, pl.ds(g * 16, 16)] = jnp.zeros((16,), jnp.float32)
        return cc

    lax.fori_loop(0, CHUNK, _zsrc, 0)
    plsc.subcore_barrier()

    def _build(sb, x):
        def _bg(g, cc):
            sv = wslab[x, pl.ds(g * 16, 16)]
            for l in range(16):
                sb[g * 16 + l, pl.ds(0, 16)] = jnp.full((16,), sv[l],
                                                        jnp.float32)
            return cc

        lax.fori_loop(0, CHUNK // 16, _bg, 0)

    _build(s0, 0)

    def _pair(q, cc):
        a = 2 * q
        b = 2 * q + 1
        pltpu.async_copy(s0, acc.at[cslab.at[a]], semw, add=True)
        _build(s1, b)
        pltpu.make_async_copy(s0, acc.at[cslab.at[a]], semw).wait()
        pltpu.async_copy(s1, acc.at[cslab.at[b]], semw, add=True)

        @pl.when(q < NCH // 2 - 1)
        def _nb():
            _build(s0, b + 1)

        pltpu.make_async_copy(s1, acc.at[cslab.at[b]], semw).wait()
        return cc

    lax.fori_loop(0, NCH // 2, _pair, 0)
    plsc.subcore_barrier()
    pltpu.sync_copy(acc.at[pl.ds(t0, TPS), :],
                    out_hbm.at[c, pl.ds(t0, TPS), :])


@functools.cache
def _make_sc_agg():
    return functools.partial(
        pl.kernel,
        out_type=jax.ShapeDtypeStruct((2, NTOT, TW), jnp.float32),
        mesh=_sc_mesh(),
        scratch_types=[
            pltpu.VMEM((CHUNK,), jnp.int32),
            pltpu.VMEM((CHUNK,), jnp.int32),
            pltpu.VMEM((CHUNK,), jnp.float32),
            pltpu.VMEM((CHUNK, TW), jnp.float32),
            pltpu.VMEM_SHARED((NTOT, TW), jnp.float32),
        ],
    )(_sc_agg_body)


def _sc_agg(row1, col1, ew1, tab, z64):
    return _make_sc_agg()(row1, col1, ew1, tab, z64)


def _sc_agg_body(row_hbm, col_hbm, ew_hbm, tab_hbm, z64_hbm, out_hbm,
                 rowbuf, colbuf, ewbuf, gbuf, acc):
    c = lax.axis_index("c")
    s = lax.axis_index("s")
    w = s * 2 + c
    t0 = pl.multiple_of(s * TPS, 8)

    pltpu.sync_copy(z64_hbm.at[pl.ds(t0, TPS), :], acc.at[pl.ds(t0, TPS), :])
    plsc.subcore_barrier()

    def _chunk(ci, carry):
        base = w * EPW + ci * CHUNK
        pltpu.sync_copy(row_hbm.at[pl.ds(base, CHUNK)], rowbuf)
        pltpu.sync_copy(col_hbm.at[pl.ds(base, CHUNK)], colbuf)
        pltpu.sync_copy(ew_hbm.at[pl.ds(base, CHUNK)], ewbuf)
        pltpu.sync_copy(tab_hbm.at[rowbuf], gbuf)

        def _scale(g, cc):
            sv = ewbuf[pl.ds(g * 16, 16)]
            for l in range(16):
                sc = sv[l]
                k = g * 16 + l
                for c2 in range(H // 16):
                    gbuf[k, pl.ds(c2 * 16, 16)] = (
                        sc * gbuf[k, pl.ds(c2 * 16, 16)])
            return cc

        lax.fori_loop(0, CHUNK // 16, _scale, 0)
        pltpu.sync_copy(gbuf, acc.at[colbuf], add=True)
        return carry

    lax.fori_loop(0, NCH, _chunk, 0)
    plsc.subcore_barrier()
    pltpu.sync_copy(acc.at[pl.ds(t0, TPS), :],
                    out_hbm.at[c, pl.ds(t0, TPS), :])


def _tc_pre_kernel(ge, ep, pe, dacc, pW, pb,
                   base0, posR, tabcat, discol):
    def renorm(t):
        n = jnp.sqrt(jnp.sum(t * t, axis=1, keepdims=True))
        return t * jnp.where(n > 1.0, 1.0 / (n + 1e-7), 1.0)

    gen = renorm(ge[...])
    m = gen.mean(axis=0)
    v = (gen * gen).mean(axis=0) - m * m
    base0[...] = jax.nn.relu((gen - m) * lax.rsqrt(v + EPS))

    epn = renorm(ep[...])
    pen = renorm(pe[...])

    deg = dacc[0, :, 0:1] + dacc[1, :, 0:1] + 1.0
    dis = lax.rsqrt(deg)
    discol[...] = dis

    tabcat[0:NGP, 0:H] = jnp.concatenate(
        [dis[0:NG] * epn, jnp.zeros((NGP - NG, H), jnp.float32)], axis=0)
    tabcat[NGP:NTOT, 0:H] = jnp.concatenate(
        [dis[NGP:NGP + NP] * pen, jnp.zeros((NPP - NP, H), jnp.float32)],
        axis=0)
    tabcat[:, H:TW] = jnp.zeros((NTOT, TW - H), jnp.float32)

    posR[...] = jnp.dot(epn, pW[...],
                        preferred_element_type=jnp.float32) + pb[0, :]


def _tc_pre(ge, ep, pe, dacc, pW, pb):
    return pl.pallas_call(
        _tc_pre_kernel,
        out_shape=[
            jax.ShapeDtypeStruct((NG, H), jnp.float32),
            jax.ShapeDtypeStruct((NG, H), jnp.float32),
            jax.ShapeDtypeStruct((NTOT, TW), jnp.float32),
            jax.ShapeDtypeStruct((NTOT, 1), jnp.float32),
        ],
    )(ge, ep, pe, dacc, pW, pb)


def _tc_mid_kernel(base0, posR, Sacc, tabcat, discol, pidx,
                   pW, pb, gW, gb, eW1, eb1, eW2, eb2,
                   fW1, fb1, fW2, fb2,
                   uA_o, uB_o, vet_o):
    Scomb = Sacc[0, :, 0:H] + Sacc[1, :, 0:H] + tabcat[:, 0:H]
    agg = discol[...] * Scomb

    pos0 = jnp.dot(agg[0:NG], pW[...],
                   preferred_element_type=jnp.float32) + pb[0, :]
    b0 = base0[...]
    beA = b0 + 0.2 * pos0
    beB = b0 + 0.2 * posR[...]

    def wbn2(hA, hB):
        m = (hA.sum(axis=0) + 31.0 * hB.sum(axis=0)) / NROWS
        sq = ((hA * hA).sum(axis=0) + 31.0 * (hB * hB).sum(axis=0)) / NROWS
        s = lax.rsqrt(sq - m * m + EPS)
        return (hA - m) * s, (hB - m) * s

    def mm(a, w, b):
        return jnp.dot(a, w, preferred_element_type=jnp.float32) + b

    hA, hB = wbn2(mm(beA, eW1[...], eb1[0, :]), mm(beB, eW1[...], eb1[0, :]))
    hA, hB = jax.nn.relu(hA), jax.nn.relu(hB)
    beA, beB = wbn2(mm(hA, eW2[...], eb2[0, :]), mm(hB, eW2[...], eb2[0, :]))

    pi = pidx[...]
    jidx = lax.broadcasted_iota(jnp.int32, (B, NPP), 1)
    M = ((jidx == pi[:, 0:1]).astype(jnp.float32)
         + (jidx == pi[:, 1:2]).astype(jnp.float32))
    aggo = agg[NGP:NTOT]
    t = jnp.dot(M, aggo, preferred_element_type=jnp.float32)
    summed = jnp.dot(t, gW[...],
                     preferred_element_type=jnp.float32) + 2.0 * gb[0, :]

    def bn(h):
        m = h.mean(axis=0)
        v = (h * h).mean(axis=0) - m * m
        return (h - m) * lax.rsqrt(v + EPS)

    et = bn(mm(jax.nn.relu(bn(mm(summed, fW1[...], fb1[0, :]))),
               fW2[...], fb2[0, :]))

    sumA, sumB = beA.sum(axis=0), beB.sum(axis=0)
    sqA, sqB = (beA * beA).sum(axis=0), (beB * beB).sum(axis=0)
    sum_et = et.sum(axis=0)
    et0 = et[0, :]
    sum_et_r = sum_et - et0
    sq_et_r = (et * et).sum(axis=0) - et0 * et0
    mean3 = (sumA + 31.0 * sumB) / NROWS + sum_et / 32.0
    Esq = (sqA + 2.0 * et0 * sumA + NG * et0 * et0
           + 31.0 * sqB + 2.0 * sum_et_r * sumB + NG * sq_et_r) / NROWS
    s3 = lax.rsqrt(Esq - mean3 * mean3 + EPS)

    uA_o[...] = (beA - mean3) * s3
    uB_o[...] = (beB - mean3) * s3
    vet_o[...] = et * s3


def _tc_mid(base0, posR, Sacc, tabcat, discol, pidx, weights):
    return pl.pallas_call(
        _tc_mid_kernel,
        out_shape=[
            jax.ShapeDtypeStruct((NG, H), jnp.float32),
            jax.ShapeDtypeStruct((NG, H), jnp.float32),
            jax.ShapeDtypeStruct((B, H), jnp.float32),
        ],
    )(base0, posR, Sacc, tabcat, discol, pidx, *weights)


def _big_kernel(uA, uB, vet, W1, b1, W2, b2, iw1,
                cgs_W1, cgs_b1, cgs_W2, cgs_b2,
                ib1c, iw2a, iw2r, ib2, xT,
                out, acc1, acc2, stats1, stats2, o1_scr):
    p = pl.program_id(0)
    j = pl.program_id(1)

    @pl.when((p == 0) & (j == 0))
    def _init():
        acc1[...] = jnp.zeros_like(acc1)
        acc2[...] = jnp.zeros_like(acc2)

    @pl.when((p == 1) & (j == 0))
    def _fin1():
        m = acc1[0, :] / NROWS
        v = acc1[1, :] / NROWS - m * m
        stats1[0, :] = m
        stats1[1, :] = jax.lax.rsqrt(v + EPS)

    @pl.when((p == 2) & (j == 0))
    def _fin2():
        m = acc2[0, :] / NROWS
        v = acc2[1, :] / NROWS - m * m
        stats2[0, :] = m
        stats2[1, :] = jax.lax.rsqrt(v + EPS)

    ua = uA[...]
    ub = uB[...]
    bidx = jax.lax.broadcasted_iota(jnp.int32, (1, B, 1), 1)
    u3 = jnp.where(bidx == 0, ua[:, None, :], ub[:, None, :])
    R = jax.nn.relu(u3 + vet[...][None, :, :]).reshape(GB * B, H)
    h1 = jnp.dot(R, W1[...], preferred_element_type=jnp.float32) + b1[0, :]

    @pl.when(p == 0)
    def _acc_h1():
        acc1[0, :] += h1.sum(axis=0)
        acc1[1, :] += (h1 * h1).sum(axis=0)

    @pl.when(p >= 1)
    def _later():
        h1n = jax.nn.relu((h1 - stats1[0, :]) * stats1[1, :])
        h2 = jnp.dot(h1n, W2[...], preferred_element_type=jnp.float32) + b2[0, :]

        @pl.when(p == 1)
        def _acc_h2():
            acc2[0, :64] += h2.sum(axis=0)
            acc2[1, :64] += (h2 * h2).sum(axis=0)

        @pl.when(p == 2)
        def _dec():
            h2n = (h2 - stats2[0, :64]) * stats2[1, :64]
            w = iw1[...]
            h3 = h2n.reshape(GB, B, H)
            o1b = (h3 * w[:, None, :]).sum(axis=-1)
            o1_scr[pl.ds(pl.multiple_of(j * GB, 8), GB), :] = o1b

            @pl.when(j == NBLK - 1)
            def _tail():
                o1 = o1_scr[...] + ib1c[...]
                z = jax.lax.dot_general(
                    o1, cgs_W1[...], (((0,), (0,)), ((), ())),
                    preferred_element_type=jnp.float32) + cgs_b1[0, :]
                zm = z.mean(axis=0)
                zv = ((z * z).mean(axis=0) - zm * zm)
                z = jax.nn.relu((z - zm) * jax.lax.rsqrt(zv + EPS))
                z = jnp.dot(z, cgs_W2[...],
                            preferred_element_type=jnp.float32) + cgs_b2[0, :]
                zm = z.mean(axis=0)
                zv = ((z * z).mean(axis=0) - zm * zm)
                cg = (z - zm) * jax.lax.rsqrt(zv + EPS)
                term2 = jax.lax.dot_general(
                    iw2r[...], cg, (((1,), (1,)), ((), ())),
                    preferred_element_type=jnp.float32)
                out[...] = (o1 * iw2a[...] + term2 + ib2[...] + xT[...])


def _run_big(uA, uB, vet, W1, b1, W2, b2, iw1,
             cgs_W1, cgs_b1, cgs_W2, cgs_b2, ib1c, iw2a, iw2r, ib2, xT):
    full = lambda s: pl.BlockSpec(s, lambda p, j: tuple(0 for _ in s))
    gblk = lambda d: pl.BlockSpec((GB, d), lambda p, j: (j, 0))
    grid = (3, NBLK)
    return pl.pallas_call(
        _big_kernel,
        grid=grid,
        in_specs=[
            gblk(H),
            gblk(H),
            full((B, H)),
            full((H, 2 * H)),
            full((1, 2 * H)),
            full((2 * H, H)),
            full((1, H)),
            gblk(H),
            full((NG, H)),
            full((1, H)),
            full((H, H)),
            full((1, H)),
            full((NG, 1)),
            full((NG, 1)),
            full((NG, H)),
            full((NG, 1)),
            full((NG, B)),
        ],
        out_specs=full((NG, B)),
        out_shape=jax.ShapeDtypeStruct((NG, B), jnp.float32),
        scratch_shapes=[
            pltpu.VMEM((2, 2 * H), jnp.float32),
            pltpu.VMEM((2, 2 * H), jnp.float32),
            pltpu.VMEM((2, 2 * H), jnp.float32),
            pltpu.VMEM((2, 2 * H), jnp.float32),
            pltpu.VMEM((NG, B), jnp.float32),
        ],
    )(uA, uB, vet, W1, b1, W2, b2, iw1,
      cgs_W1, cgs_b1, cgs_W2, cgs_b2, ib1c, iw2a, iw2r, ib2, xT)


def _pad_to(a, n, val):
    return jnp.concatenate([a, jnp.full((n - a.shape[0],), val, a.dtype)])


def _pad_spread(a, n):
    k = n - a.shape[0]
    return jnp.concatenate([a, (jnp.arange(k, dtype=jnp.int32) * 7) % NTOT])


def kernel(x, params, pert_idx, batch, G_coexpress, G_coexpress_weight,
           G_go, G_go_weight):
    p = params
    gcw = G_coexpress_weight
    ggw = G_go_weight

    rowc = _pad_to(jnp.concatenate(
        [G_coexpress[0].astype(jnp.int32),
         G_go[0].astype(jnp.int32) + NGP]), ETOT, 0).reshape(NW, NCH, CHUNK)
    colc = _pad_spread(jnp.concatenate(
        [G_coexpress[1].astype(jnp.int32),
         G_go[1].astype(jnp.int32) + NGP]), ETOT).reshape(NW, NCH, CHUNK)
    ewc = _pad_to(jnp.concatenate(
        [gcw, ggw]), ETOT, 0.0).reshape(NW, NCH, CHUNK)

    z64 = jnp.zeros((NTOT, TW), jnp.float32)

    dacc = _sc_deg(colc, ewc, z64)
    base0, posR, tabcat, discol = _tc_pre(
        p['gene_emb'], p['emb_pos'], p['pert_emb'], dacc,
        p['lin_pos_W'], p['lin_pos_b'].reshape(1, H))
    Sacc = _sc_agg(rowc.reshape(ETOT), colc.reshape(ETOT),
                   ewc.reshape(ETOT), tabcat, z64)
    uA, uB, vet = _tc_mid(
        base0, posR, Sacc, tabcat, discol, pert_idx.astype(jnp.int32),
        (p['lin_pos_W'], p['lin_pos_b'].reshape(1, H),
         p['lin_go_W'], p['lin_go_b'].reshape(1, H),
         p['etv2_W1'], p['etv2_b1'].reshape(1, H),
         p['etv2_W2'], p['etv2_b2'].reshape(1, H),
         p['pf_W1'], p['pf_b1'].reshape(1, H),
         p['pf_W2'], p['pf_b2'].reshape(1, H)))

    outT = _run_big(
        uA, uB, vet,
        p['rw_W1'], p['rw_b1'].reshape(1, 2 * H),
        p['rw_W2'], p['rw_b2'].reshape(1, H),
        p['indv_w1'][:, :, 0],
        p['cgs_W1'], p['cgs_b1'].reshape(1, H),
        p['cgs_W2'], p['cgs_b2'].reshape(1, H),
        p['indv_b1'],
        p['indv_w2'][0, :, 0].reshape(NG, 1),
        p['indv_w2'][0, :, 1:],
        p['indv_b2'][0].reshape(NG, 1),
        x.reshape(B, NG).T,
    )
    return outT.T


# speedup vs baseline: 1.2466x; 1.2466x over previous
import functools
import jax
import jax.numpy as jnp
from jax import lax
from jax.experimental import pallas as pl
from jax.experimental.pallas import tpu as pltpu
from jax.experimental.pallas import tpu_sc as plsc

NG = 5000
NP = 2000
H = 64
B = 32
EPS = 1e-5
GB = 200
NBLK = NG // GB
NROWS = float(B * NG)

NGP = 5120
NPP = 2048
NTOT = NGP + NPP
DUMP = NTOT - 1
ECO = 100000
EGO = 40000
NW = 32
CHUNK = 128
ECO_P = 102400
EGO_P = 40960
ETOT = ECO_P + EGO_P
EPW = ETOT // NW
NCH = EPW // CHUNK
TPS = NTOT // 16
TW = 128

def _sc_mesh():
    return plsc.VectorSubcoreMesh(core_axis_name="c", subcore_axis_name="s")


@functools.cache
def _make_sc_deg():
    return functools.partial(
        pl.kernel,
        out_type=jax.ShapeDtypeStruct((2, NTOT, TW), jnp.float32),
        mesh=_sc_mesh(),
        scratch_types=[
            pltpu.VMEM((NCH, CHUNK), jnp.int32),
            pltpu.VMEM((NCH, CHUNK), jnp.float32),
            pltpu.VMEM((CHUNK, TW), jnp.float32),
            pltpu.VMEM((CHUNK, TW), jnp.float32),
            pltpu.VMEM_SHARED((NTOT, TW), jnp.float32),
            pltpu.SemaphoreType.DMA,
        ],
    )(_sc_deg_body)


def _sc_deg(col3, ew3, z64):
    return _make_sc_deg()(col3, ew3, z64)


def _sc_deg_body(col_hbm, ew_hbm, z64_hbm, out_hbm,
                 cslab, wslab, s0, s1, acc, semw):
    c = lax.axis_index("c")
    s = lax.axis_index("s")
    w = s * 2 + c
    t0 = pl.multiple_of(s * TPS, 8)

    pltpu.sync_copy(z64_hbm.at[pl.ds(t0, TPS), :], acc.at[pl.ds(t0, TPS), :])
    pltpu.sync_copy(col_hbm.at[w], cslab)
    pltpu.sync_copy(ew_hbm.at[w], wslab)

    def _zsrc(k, cc):
        for g in range(1, TW // 16):
            s0[k, pl.ds(g * 16, 16)] = jnp.zeros((16,), jnp.float32)
            s1[k, pl.ds(g * 16, 16)] = jnp.zeros((16,), jnp.float32)
        return cc

    lax.fori_loop(0, CHUNK, _zsrc, 0)
    plsc.subcore_barrier()

    def _build(sb, x):
        def _bg(g, cc):
            sv = wslab[x, pl.ds(g * 16, 16)]
            for l in range(16):
                sb[g * 16 + l, pl.ds(0, 16)] = jnp.full((16,), sv[l],
                                                        jnp.float32)
            return cc

        lax.fori_loop(0, CHUNK // 16, _bg, 0)

    _build(s0, 0)

    def _pair(q, cc):
        a = 2 * q
        b = 2 * q + 1
        pltpu.async_copy(s0, acc.at[cslab.at[a]], semw, add=True)
        _build(s1, b)
        pltpu.make_async_copy(s0, acc.at[cslab.at[a]], semw).wait()
        pltpu.async_copy(s1, acc.at[cslab.at[b]], semw, add=True)

        @pl.when(q < NCH // 2 - 1)
        def _nb():
            _build(s0, b + 1)

        pltpu.make_async_copy(s1, acc.at[cslab.at[b]], semw).wait()
        return cc

    lax.fori_loop(0, NCH // 2, _pair, 0)
    if NCH % 2:
        _build(s0, NCH - 1)
        pltpu.sync_copy(s0, acc.at[cslab.at[NCH - 1]], add=True)
    plsc.subcore_barrier()
    pltpu.sync_copy(acc.at[pl.ds(t0, TPS), :],
                    out_hbm.at[c, pl.ds(t0, TPS), :])


@functools.cache
def _make_sc_agg():
    return functools.partial(
        pl.kernel,
        out_type=jax.ShapeDtypeStruct((2, NTOT, TW), jnp.float32),
        mesh=_sc_mesh(),
        scratch_types=[
            pltpu.VMEM((CHUNK,), jnp.int32),
            pltpu.VMEM((CHUNK,), jnp.int32),
            pltpu.VMEM((CHUNK,), jnp.float32),
            pltpu.VMEM((CHUNK, TW), jnp.float32),
            pltpu.VMEM_SHARED((NTOT, TW), jnp.float32),
        ],
    )(_sc_agg_body)


def _sc_agg(row1, col1, ew1, tab, z64):
    return _make_sc_agg()(row1, col1, ew1, tab, z64)


def _sc_agg_body(row_hbm, col_hbm, ew_hbm, tab_hbm, z64_hbm, out_hbm,
                 rowbuf, colbuf, ewbuf, gbuf, acc):
    c = lax.axis_index("c")
    s = lax.axis_index("s")
    w = s * 2 + c
    t0 = pl.multiple_of(s * TPS, 8)

    pltpu.sync_copy(z64_hbm.at[pl.ds(t0, TPS), :], acc.at[pl.ds(t0, TPS), :])
    plsc.subcore_barrier()

    def _chunk(ci, carry):
        base = w * EPW + ci * CHUNK
        pltpu.sync_copy(row_hbm.at[pl.ds(base, CHUNK)], rowbuf)
        pltpu.sync_copy(col_hbm.at[pl.ds(base, CHUNK)], colbuf)
        pltpu.sync_copy(ew_hbm.at[pl.ds(base, CHUNK)], ewbuf)
        pltpu.sync_copy(tab_hbm.at[rowbuf], gbuf)

        def _scale(g, cc):
            sv = ewbuf[pl.ds(g * 16, 16)]
            for l in range(16):
                sc = sv[l]
                k = g * 16 + l
                for c2 in range(H // 16):
                    gbuf[k, pl.ds(c2 * 16, 16)] = (
                        sc * gbuf[k, pl.ds(c2 * 16, 16)])
            return cc

        lax.fori_loop(0, CHUNK // 16, _scale, 0)
        pltpu.sync_copy(gbuf, acc.at[colbuf], add=True)
        return carry

    lax.fori_loop(0, NCH, _chunk, 0)
    plsc.subcore_barrier()
    pltpu.sync_copy(acc.at[pl.ds(t0, TPS), :],
                    out_hbm.at[c, pl.ds(t0, TPS), :])


def _tc_pre_kernel(ge, ep, pe, dacc, pW, pb,
                   base0, posR, tabcat, discol):
    def renorm(t):
        n = jnp.sqrt(jnp.sum(t * t, axis=1, keepdims=True))
        return t * jnp.where(n > 1.0, 1.0 / (n + 1e-7), 1.0)

    gen = renorm(ge[...])
    m = gen.mean(axis=0)
    v = (gen * gen).mean(axis=0) - m * m
    base0[...] = jax.nn.relu((gen - m) * lax.rsqrt(v + EPS))

    epn = renorm(ep[...])
    pen = renorm(pe[...])

    deg = dacc[0, :, 0:1] + dacc[1, :, 0:1] + 1.0
    dis = lax.rsqrt(deg)
    discol[...] = dis

    tabcat[0:NGP, 0:H] = jnp.concatenate(
        [dis[0:NG] * epn, jnp.zeros((NGP - NG, H), jnp.float32)], axis=0)
    tabcat[NGP:NTOT, 0:H] = jnp.concatenate(
        [dis[NGP:NGP + NP] * pen, jnp.zeros((NPP - NP, H), jnp.float32)],
        axis=0)
    tabcat[:, H:TW] = jnp.zeros((NTOT, TW - H), jnp.float32)

    posR[...] = jnp.dot(epn, pW[...],
                        preferred_element_type=jnp.float32) + pb[0, :]


def _tc_pre(ge, ep, pe, dacc, pW, pb):
    return pl.pallas_call(
        _tc_pre_kernel,
        out_shape=[
            jax.ShapeDtypeStruct((NG, H), jnp.float32),
            jax.ShapeDtypeStruct((NG, H), jnp.float32),
            jax.ShapeDtypeStruct((NTOT, TW), jnp.float32),
            jax.ShapeDtypeStruct((NTOT, 1), jnp.float32),
        ],
    )(ge, ep, pe, dacc, pW, pb)


def _tc_mid_kernel(base0, posR, Sacc, tabcat, discol, pidx,
                   pW, pb, gW, gb, eW1, eb1, eW2, eb2,
                   fW1, fb1, fW2, fb2,
                   uA_o, uB_o, vet_o):
    Scomb = Sacc[0, :, 0:H] + Sacc[1, :, 0:H] + tabcat[:, 0:H]
    agg = discol[...] * Scomb

    pos0 = jnp.dot(agg[0:NG], pW[...],
                   preferred_element_type=jnp.float32) + pb[0, :]
    b0 = base0[...]
    beA = b0 + 0.2 * pos0
    beB = b0 + 0.2 * posR[...]

    def wbn2(hA, hB):
        m = (hA.sum(axis=0) + 31.0 * hB.sum(axis=0)) / NROWS
        sq = ((hA * hA).sum(axis=0) + 31.0 * (hB * hB).sum(axis=0)) / NROWS
        s = lax.rsqrt(sq - m * m + EPS)
        return (hA - m) * s, (hB - m) * s

    def mm(a, w, b):
        return jnp.dot(a, w, preferred_element_type=jnp.float32) + b

    hA, hB = wbn2(mm(beA, eW1[...], eb1[0, :]), mm(beB, eW1[...], eb1[0, :]))
    hA, hB = jax.nn.relu(hA), jax.nn.relu(hB)
    beA, beB = wbn2(mm(hA, eW2[...], eb2[0, :]), mm(hB, eW2[...], eb2[0, :]))

    pi = pidx[...]
    jidx = lax.broadcasted_iota(jnp.int32, (B, NPP), 1)
    M = ((jidx == pi[:, 0:1]).astype(jnp.float32)
         + (jidx == pi[:, 1:2]).astype(jnp.float32))
    aggo = agg[NGP:NTOT]
    t = jnp.dot(M, aggo, preferred_element_type=jnp.float32)
    summed = jnp.dot(t, gW[...],
                     preferred_element_type=jnp.float32) + 2.0 * gb[0, :]

    def bn(h):
        m = h.mean(axis=0)
        v = (h * h).mean(axis=0) - m * m
        return (h - m) * lax.rsqrt(v + EPS)

    et = bn(mm(jax.nn.relu(bn(mm(summed, fW1[...], fb1[0, :]))),
               fW2[...], fb2[0, :]))

    sumA, sumB = beA.sum(axis=0), beB.sum(axis=0)
    sqA, sqB = (beA * beA).sum(axis=0), (beB * beB).sum(axis=0)
    sum_et = et.sum(axis=0)
    et0 = et[0, :]
    sum_et_r = sum_et - et0
    sq_et_r = (et * et).sum(axis=0) - et0 * et0
    mean3 = (sumA + 31.0 * sumB) / NROWS + sum_et / 32.0
    Esq = (sqA + 2.0 * et0 * sumA + NG * et0 * et0
           + 31.0 * sqB + 2.0 * sum_et_r * sumB + NG * sq_et_r) / NROWS
    s3 = lax.rsqrt(Esq - mean3 * mean3 + EPS)

    uA_o[...] = (beA - mean3) * s3
    uB_o[...] = (beB - mean3) * s3
    vet_o[...] = et * s3


def _tc_mid(base0, posR, Sacc, tabcat, discol, pidx, weights):
    return pl.pallas_call(
        _tc_mid_kernel,
        out_shape=[
            jax.ShapeDtypeStruct((NG, H), jnp.float32),
            jax.ShapeDtypeStruct((NG, H), jnp.float32),
            jax.ShapeDtypeStruct((B, H), jnp.float32),
        ],
    )(base0, posR, Sacc, tabcat, discol, pidx, *weights)


def _big_kernel(uA, uB, vet, W1, b1, W2, b2, iw1,
                cgs_W1, cgs_b1, cgs_W2, cgs_b2,
                ib1c, iw2a, iw2r, ib2, xT,
                out, acc1, acc2, stats1, stats2, o1_scr):
    p = pl.program_id(0)
    j = pl.program_id(1)

    @pl.when((p == 0) & (j == 0))
    def _init():
        acc1[...] = jnp.zeros_like(acc1)
        acc2[...] = jnp.zeros_like(acc2)

    @pl.when((p == 1) & (j == 0))
    def _fin1():
        m = acc1[0, :] / NROWS
        v = acc1[1, :] / NROWS - m * m
        stats1[0, :] = m
        stats1[1, :] = jax.lax.rsqrt(v + EPS)

    @pl.when((p == 2) & (j == 0))
    def _fin2():
        m = acc2[0, :] / NROWS
        v = acc2[1, :] / NROWS - m * m
        stats2[0, :] = m
        stats2[1, :] = jax.lax.rsqrt(v + EPS)

    ua = uA[...]
    ub = uB[...]
    bidx = jax.lax.broadcasted_iota(jnp.int32, (1, B, 1), 1)
    u3 = jnp.where(bidx == 0, ua[:, None, :], ub[:, None, :])
    R = jax.nn.relu(u3 + vet[...][None, :, :]).reshape(GB * B, H)
    h1 = jnp.dot(R, W1[...], preferred_element_type=jnp.float32) + b1[0, :]

    @pl.when(p == 0)
    def _acc_h1():
        acc1[0, :] += h1.sum(axis=0)
        acc1[1, :] += (h1 * h1).sum(axis=0)

    @pl.when(p >= 1)
    def _later():
        h1n = jax.nn.relu((h1 - stats1[0, :]) * stats1[1, :])
        h2 = jnp.dot(h1n, W2[...], preferred_element_type=jnp.float32) + b2[0, :]

        @pl.when(p == 1)
        def _acc_h2():
            acc2[0, :64] += h2.sum(axis=0)
            acc2[1, :64] += (h2 * h2).sum(axis=0)

        @pl.when(p == 2)
        def _dec():
            h2n = (h2 - stats2[0, :64]) * stats2[1, :64]
            w = iw1[...]
            h3 = h2n.reshape(GB, B, H)
            o1b = (h3 * w[:, None, :]).sum(axis=-1)
            o1_scr[pl.ds(pl.multiple_of(j * GB, 8), GB), :] = o1b

            @pl.when(j == NBLK - 1)
            def _tail():
                o1 = o1_scr[...] + ib1c[...]
                z = jax.lax.dot_general(
                    o1, cgs_W1[...], (((0,), (0,)), ((), ())),
                    preferred_element_type=jnp.float32) + cgs_b1[0, :]
                zm = z.mean(axis=0)
                zv = ((z * z).mean(axis=0) - zm * zm)
                z = jax.nn.relu((z - zm) * jax.lax.rsqrt(zv + EPS))
                z = jnp.dot(z, cgs_W2[...],
                            preferred_element_type=jnp.float32) + cgs_b2[0, :]
                zm = z.mean(axis=0)
                zv = ((z * z).mean(axis=0) - zm * zm)
                cg = (z - zm) * jax.lax.rsqrt(zv + EPS)
                term2 = jax.lax.dot_general(
                    iw2r[...], cg, (((1,), (1,)), ((), ())),
                    preferred_element_type=jnp.float32)
                out[...] = (o1 * iw2a[...] + term2 + ib2[...] + xT[...])


def _run_big(uA, uB, vet, W1, b1, W2, b2, iw1,
             cgs_W1, cgs_b1, cgs_W2, cgs_b2, ib1c, iw2a, iw2r, ib2, xT):
    full = lambda s: pl.BlockSpec(s, lambda p, j: tuple(0 for _ in s))
    gblk = lambda d: pl.BlockSpec((GB, d), lambda p, j: (j, 0))
    grid = (3, NBLK)
    return pl.pallas_call(
        _big_kernel,
        grid=grid,
        in_specs=[
            gblk(H),
            gblk(H),
            full((B, H)),
            full((H, 2 * H)),
            full((1, 2 * H)),
            full((2 * H, H)),
            full((1, H)),
            gblk(H),
            full((NG, H)),
            full((1, H)),
            full((H, H)),
            full((1, H)),
            full((NG, 1)),
            full((NG, 1)),
            full((NG, H)),
            full((NG, 1)),
            full((NG, B)),
        ],
        out_specs=full((NG, B)),
        out_shape=jax.ShapeDtypeStruct((NG, B), jnp.float32),
        scratch_shapes=[
            pltpu.VMEM((2, 2 * H), jnp.float32),
            pltpu.VMEM((2, 2 * H), jnp.float32),
            pltpu.VMEM((2, 2 * H), jnp.float32),
            pltpu.VMEM((2, 2 * H), jnp.float32),
            pltpu.VMEM((NG, B), jnp.float32),
        ],
    )(uA, uB, vet, W1, b1, W2, b2, iw1,
      cgs_W1, cgs_b1, cgs_W2, cgs_b2, ib1c, iw2a, iw2r, ib2, xT)


def _pad_to(a, n, val):
    return jnp.concatenate([a, jnp.full((n - a.shape[0],), val, a.dtype)])


def _pad_spread(a, n):
    k = n - a.shape[0]
    return jnp.concatenate([a, (jnp.arange(k, dtype=jnp.int32) * 7) % NTOT])


def kernel(x, params, pert_idx, batch, G_coexpress, G_coexpress_weight,
           G_go, G_go_weight):
    p = params
    gcw = G_coexpress_weight
    ggw = G_go_weight

    rowc = jnp.concatenate(
        [_pad_to(G_coexpress[0].astype(jnp.int32), ECO_P, 0),
         _pad_to(G_go[0].astype(jnp.int32) + NGP, EGO_P, 0)])
    colc = jnp.concatenate(
        [_pad_spread(G_coexpress[1].astype(jnp.int32), ECO_P),
         _pad_spread(G_go[1].astype(jnp.int32) + NGP, EGO_P)])
    ewc = jnp.concatenate([_pad_to(gcw, ECO_P, 0.0),
                           _pad_to(ggw, EGO_P, 0.0)])

    z64 = jnp.zeros((NTOT, TW), jnp.float32)

    dacc = _sc_deg(colc.reshape(NW, NCH, CHUNK),
                   ewc.reshape(NW, NCH, CHUNK), z64)
    base0, posR, tabcat, discol = _tc_pre(
        p['gene_emb'], p['emb_pos'], p['pert_emb'], dacc,
        p['lin_pos_W'], p['lin_pos_b'].reshape(1, H))
    Sacc = _sc_agg(rowc, colc, ewc, tabcat, z64)
    uA, uB, vet = _tc_mid(
        base0, posR, Sacc, tabcat, discol, pert_idx.astype(jnp.int32),
        (p['lin_pos_W'], p['lin_pos_b'].reshape(1, H),
         p['lin_go_W'], p['lin_go_b'].reshape(1, H),
         p['etv2_W1'], p['etv2_b1'].reshape(1, H),
         p['etv2_W2'], p['etv2_b2'].reshape(1, H),
         p['pf_W1'], p['pf_b1'].reshape(1, H),
         p['pf_W2'], p['pf_b2'].reshape(1, H)))

    outT = _run_big(
        uA, uB, vet,
        p['rw_W1'], p['rw_b1'].reshape(1, 2 * H),
        p['rw_W2'], p['rw_b2'].reshape(1, H),
        p['indv_w1'][:, :, 0],
        p['cgs_W1'], p['cgs_b1'].reshape(1, H),
        p['cgs_W2'], p['cgs_b2'].reshape(1, H),
        p['indv_b1'],
        p['indv_w2'][0, :, 0].reshape(NG, 1),
        p['indv_w2'][0, :, 1:],
        p['indv_b2'][0].reshape(NG, 1),
        x.reshape(B, NG).T,
    )
    return outT.T
